# baseline (speedup 1.0000x reference)
import functools
import jax
import jax.numpy as jnp
from jax.experimental import pallas as pl

K = 40
EPS = 1e-5
N = 2048
BR = 256


def _sortable(pd):
    bbits = jax.lax.bitcast_convert_type(pd, jnp.int32)
    return jnp.where(bbits >= 0, bbits, (~bbits) ^ jnp.int32(-2147483648))


def _pd_body(ft_blk, ft_all, out_ref, t_ref, g_ref):
    a = ft_blk[0]
    t = ft_all[0]
    ab = a.astype(jnp.bfloat16)
    tb = t.astype(jnp.bfloat16)
    dot = jax.lax.dot_general(ab, tb, (((1,), (1,)), ((), ())),
                              preferred_element_type=jnp.float32)
    inner = -2.0 * dot
    xx_r = jnp.sum(a * a, axis=1, keepdims=True)
    xx_c = jnp.sum(t * t, axis=1, keepdims=True).T
    pd = (-xx_r - inner) - xx_c
    out_ref[0] = pd

    ki = _sortable(pd)
    lo = jnp.min(ki, axis=1, keepdims=True) - 1
    hi = jnp.max(ki, axis=1, keepdims=True)

    def body(_, carry):
        lo, hi = carry
        mid = lo + jax.lax.shift_right_logical(hi - lo, 1)
        cnt = jnp.sum((ki > mid).astype(jnp.int32), axis=1, keepdims=True)
        ge = cnt >= K
        return jnp.where(ge, mid, lo), jnp.where(ge, hi, mid)

    lo, hi = jax.lax.fori_loop(0, 32, body, (lo, hi))
    tkey = hi
    gcnt = jnp.sum((ki > tkey).astype(jnp.int32), axis=1, keepdims=True)
    t_ref[0, 0] = tkey
    g_ref[0, 0] = gcnt


def _pd_pallas(fT):
    B, n, C = fT.shape
    nb = n // BR
    pd, tk, gc = pl.pallas_call(
        _pd_body,
        grid=(B, nb),
        in_specs=[
            pl.BlockSpec((1, BR, C), lambda b, r: (b, r, 0)),
            pl.BlockSpec((1, n, C), lambda b, r: (b, 0, 0)),
        ],
        out_specs=[
            pl.BlockSpec((1, BR, n), lambda b, r: (b, r, 0)),
            pl.BlockSpec((1, 1, BR, 1), lambda b, r: (b, r, 0, 0)),
            pl.BlockSpec((1, 1, BR, 1), lambda b, r: (b, r, 0, 0)),
        ],
        out_shape=[
            jax.ShapeDtypeStruct((B, n, n), jnp.float32),
            jax.ShapeDtypeStruct((B, nb, BR, 1), jnp.int32),
            jax.ShapeDtypeStruct((B, nb, BR, 1), jnp.int32),
        ],
    )(fT, fT)
    return pd, tk.reshape(B, n), gc.reshape(B, n)


def _edge_layer(fT, W, g, b):
    B, n, C = fT.shape
    pd, tkey, gcnt = _pd_pallas(fT)
    ki = _sortable(pd)
    mask_gt = ki > tkey[:, :, None]
    mask_eq = ki == tkey[:, :, None]
    eqrank = jnp.cumsum(mask_eq.astype(jnp.int32), axis=-1)
    mask = mask_gt | (mask_eq & (eqrank <= (K - gcnt)[:, :, None]))
    iota = jnp.arange(n, dtype=jnp.int32)
    scores = jnp.where(mask, n - iota, -1)
    idx = jax.lax.top_k(scores, K)[1]
    fj = jax.vmap(lambda ff, ii: ff[ii])(fT, idx)
    G = fj - fT[:, :, None, :]
    X = jnp.concatenate([G, jnp.broadcast_to(fT[:, :, None, :], G.shape)], axis=-1)
    y = jnp.einsum('bnkc,oc->bnko', X, W)
    mean = jnp.mean(y, axis=(0, 1, 2), keepdims=True)
    var = jnp.var(y, axis=(0, 1, 2), keepdims=True)
    yn = (y - mean) / jnp.sqrt(var + EPS) * g + b
    yn = jnp.where(yn > 0, yn, 0.2 * yn)
    return jnp.max(yn, axis=2)


def kernel(x, fx, W1, g1, b1, W2, g2, b2, W3, g3, b3, W4, g4, b4, W5, g5, b5, Wp1, bp1, Wp2, bp2):
    h = jnp.concatenate([x, fx], axis=-1)
    x1 = _edge_layer(h, W1, g1, b1)
    x2 = _edge_layer(x1, W2, g2, b2)
    x3 = _edge_layer(x2, W3, g3, b3)
    x4 = _edge_layer(x3, W4, g4, b4)
    hc = jnp.concatenate([x1, x2, x3, x4], axis=-1)
    e = hc @ W5.T
    mean = jnp.mean(e, axis=(0, 1))
    var = jnp.var(e, axis=(0, 1))
    en = (e - mean) / jnp.sqrt(var + EPS) * g5 + b5
    en = jnp.where(en > 0, en, 0.2 * en)
    p = jnp.maximum(en @ Wp1.T + bp1, 0.0)
    return p @ Wp2.T + bp2


# speedup vs baseline: 3.8989x; 3.8989x over previous
import functools

import jax
import jax.numpy as jnp
from jax import lax
from jax.experimental import pallas as pl
from jax.experimental.pallas import tpu as pltpu
from jax.experimental.pallas import tpu_sc as plsc

K = 40
EPS = 1e-5
N = 2048
B = 4
BN = B * N
KP = 48
GP = 128
NW = 32
RPW = BN // NW
BR = 256
BRC = 256
IMIN = -2147483648


def _sortable(x):
    bbits = lax.bitcast_convert_type(x, jnp.int32)
    return jnp.where(bbits >= 0, bbits, (~bbits) ^ jnp.int32(IMIN))


def _pd_body(ft_blk, ft_all, out_ref, t_ref, g_ref):
    a = ft_blk[0]
    t = ft_all[0]
    dot = lax.dot_general(a.astype(jnp.bfloat16), t.astype(jnp.bfloat16),
                          (((1,), (1,)), ((), ())),
                          preferred_element_type=jnp.float32)
    inner = -2.0 * dot
    xx_r = jnp.sum(a * a, axis=1, keepdims=True)
    xx_c = jnp.sum(t * t, axis=1, keepdims=True).T
    pd = (-xx_r - inner) - xx_c
    out_ref[0] = pd

    ki = _sortable(pd)
    lo = jnp.min(ki, axis=1, keepdims=True) - 1
    hi = jnp.max(ki, axis=1, keepdims=True)

    def body(_, carry):
        lo, hi = carry
        mid = lo + lax.shift_right_logical(hi - lo, 1)
        cnt = jnp.sum((ki > mid).astype(jnp.int32), axis=1, keepdims=True)
        ge = cnt >= K
        return jnp.where(ge, mid, lo), jnp.where(ge, hi, mid)

    lo, hi = lax.fori_loop(0, 32, body, (lo, hi))
    t_ref[0, 0] = hi
    g_ref[0, 0] = jnp.sum((ki > hi).astype(jnp.int32), axis=1, keepdims=True)


def _pd_pallas(fT):
    _, n, C = fT.shape
    nb = n // BR
    pd, tk, gc = pl.pallas_call(
        _pd_body,
        grid=(B, nb),
        in_specs=[
            pl.BlockSpec((1, BR, C), lambda b, r: (b, r, 0)),
            pl.BlockSpec((1, n, C), lambda b, r: (b, 0, 0)),
        ],
        out_specs=[
            pl.BlockSpec((1, BR, n), lambda b, r: (b, r, 0)),
            pl.BlockSpec((1, 1, BR, 1), lambda b, r: (b, r, 0, 0)),
            pl.BlockSpec((1, 1, BR, 1), lambda b, r: (b, r, 0, 0)),
        ],
        out_shape=[
            jax.ShapeDtypeStruct((B, n, n), jnp.float32),
            jax.ShapeDtypeStruct((B, nb, BR, 1), jnp.int32),
            jax.ShapeDtypeStruct((B, nb, BR, 1), jnp.int32),
        ],
    )(fT, fT)
    return (pd.reshape(BN, n), tk.reshape(BN), gc.reshape(BN))


def _make_sc_select_gather(CP):
    mesh = plsc.VectorSubcoreMesh(core_axis_name="c", subcore_axis_name="s")

    @functools.partial(
        pl.kernel, mesh=mesh,
        compiler_params=pltpu.CompilerParams(needs_layout_passes=False),
        out_type=jax.ShapeDtypeStruct((BN * K, GP), jnp.float32),
        scratch_types=[
            pltpu.VMEM((N,), jnp.float32),
            pltpu.VMEM((RPW,), jnp.int32),
            pltpu.VMEM((RPW,), jnp.int32),
            pltpu.VMEM((KP,), jnp.int32),
            pltpu.VMEM((KP, GP), jnp.float32),
            pltpu.SemaphoreType.DMA,
        ],
    )
    def sck(pd_hbm, tk_hbm, g_hbm, ft_hbm, out_hbm, pdrow, tkw, gw, idxb, fjb, sem):
        wid = lax.axis_index("s") * 2 + lax.axis_index("c")
        base = wid * RPW
        pltpu.sync_copy(tk_hbm.at[pl.ds(base, RPW)], tkw)
        pltpu.sync_copy(g_hbm.at[pl.ds(base, RPW)], gw)
        iota = lax.iota(jnp.int32, 16)

        def row_body(i, carry):
            r = base + i
            pltpu.sync_copy(pd_hbm.at[r], pdrow)
            isplat = jnp.full((16,), i, dtype=jnp.int32)
            tk_v = plsc.load_gather(tkw, [isplat])
            g_v = plsc.load_gather(gw, [isplat])
            need = K - g_v
            b_off = jnp.bitwise_and(r, jnp.int32(-N))
            zero16 = jnp.zeros((16,), jnp.int32)

            def chunk_body(cidx, cc):
                sel_cnt, eq_cnt = cc
                v = pdrow[pl.ds(cidx * 16, 16)]
                ki = _sortable(v)
                m_gt = ki > tk_v
                m_eq = ki == tk_v
                eq_rank = plsc.cumsum(m_eq.astype(jnp.int32)) + eq_cnt
                m = jnp.logical_or(m_gt, jnp.logical_and(m_eq, eq_rank <= need))
                mi = m.astype(jnp.int32)
                pos = plsc.cumsum(mi) + (sel_cnt - 1)
                vals = iota + (cidx * 16 + b_off)
                plsc.store_scatter(idxb, [pos], vals, mask=m)
                return (sel_cnt + plsc.all_reduce_population_count(m),
                        eq_cnt + plsc.all_reduce_population_count(m_eq))

            lax.fori_loop(0, N // 16, chunk_body, (zero16, zero16))

            plsc.store_scatter(idxb, [iota + K], jnp.full((16,), r, dtype=jnp.int32),
                               mask=iota < (KP - K))
            pltpu.async_copy(ft_hbm.at[idxb], fjb, sem).wait()

            for j in range(K):
                for ch in range(CP // 16):
                    sl = pl.ds(ch * 16, 16)
                    fjb[j, sl] = fjb[j, sl] - fjb[K, sl]
            pltpu.sync_copy(fjb.at[pl.ds(0, K)], out_hbm.at[pl.ds(r * K, K)])
            return carry

        lax.fori_loop(0, RPW, row_body, jnp.int32(0))

    return sck


def _df_acc(acc_ref, hi_i, lo_i, s):
    a = acc_ref[hi_i:hi_i + 1, :]
    l = acc_ref[lo_i:lo_i + 1, :]
    t = a + s
    bb = t - a
    err = (a - (t - bb)) + (s - bb)
    acc_ref[hi_i:hi_i + 1, :] = t
    acc_ref[lo_i:lo_i + 1, :] = l + err


def _make_conv(C, H):
    def body(g_blk, fi_blk, w_ref, m_ref, mn_ref, acc_ref):
        gb = g_blk[:, :C].astype(jnp.bfloat16)
        fib = fi_blk[:, :C]
        fi3 = jnp.broadcast_to(fib[:, None, :], (BRC, K, C)) \
                 .reshape(BRC * K, C).astype(jnp.bfloat16)
        lhs = jnp.concatenate([gb, fi3], axis=1)
        wb = w_ref[...].astype(jnp.bfloat16)
        y = lax.dot_general(lhs, wb, (((1,), (1,)), ((), ())),
                            preferred_element_type=jnp.float32)
        y3 = y.reshape(BRC, K, H)
        m_ref[...] = jnp.max(y3, axis=1)
        mn_ref[...] = jnp.min(y3, axis=1)

        @pl.when(pl.program_id(0) == 0)
        def _init():
            acc_ref[...] = jnp.zeros((4, H), jnp.float32)

        _df_acc(acc_ref, 0, 1, jnp.sum(y, axis=0, keepdims=True))
        _df_acc(acc_ref, 2, 3, jnp.sum(y * y, axis=0, keepdims=True))

    def run(G, fi, W):
        H2 = W.shape[1]
        return pl.pallas_call(
            body,
            grid=(BN // BRC,),
            in_specs=[
                pl.BlockSpec((BRC * K, GP), lambda r: (r, 0)),
                pl.BlockSpec((BRC, GP), lambda r: (r, 0)),
                pl.BlockSpec((H, H2), lambda r: (0, 0)),
            ],
            out_specs=[
                pl.BlockSpec((BRC, H), lambda r: (r, 0)),
                pl.BlockSpec((BRC, H), lambda r: (r, 0)),
                pl.BlockSpec((4, H), lambda r: (0, 0)),
            ],
            out_shape=[
                jax.ShapeDtypeStruct((BN, H), jnp.float32),
                jax.ShapeDtypeStruct((BN, H), jnp.float32),
                jax.ShapeDtypeStruct((4, H), jnp.float32),
            ],
        )(G, fi, W)

    return run


def _fin_body(m_ref, mn_ref, acc_ref, gam_ref, bet_ref, out_ref, *, cnt):
    s1 = acc_ref[0:1, :] + acc_ref[1:2, :]
    mean = s1 / cnt
    var = (acc_ref[2:3, :] / cnt - mean * mean) + acc_ref[3:4, :] / cnt
    rs = jnp.sqrt(var + EPS)
    gam = gam_ref[...]
    sel = jnp.where(gam >= 0, m_ref[...], mn_ref[...])
    yn = (sel - mean) / rs * gam + bet_ref[...]
    out_ref[...] = jnp.where(yn > 0, yn, 0.2 * yn)


def _finalize(M, mn, acc, gam, bet, cnt):
    H = M.shape[1]
    return pl.pallas_call(
        functools.partial(_fin_body, cnt=cnt),
        grid=(BN // BRC,),
        in_specs=[
            pl.BlockSpec((BRC, H), lambda r: (r, 0)),
            pl.BlockSpec((BRC, H), lambda r: (r, 0)),
            pl.BlockSpec((4, H), lambda r: (0, 0)),
            pl.BlockSpec((1, H), lambda r: (0, 0)),
            pl.BlockSpec((1, H), lambda r: (0, 0)),
        ],
        out_specs=pl.BlockSpec((BRC, H), lambda r: (r, 0)),
        out_shape=jax.ShapeDtypeStruct((BN, H), jnp.float32),
    )(M, mn, acc, gam.reshape(1, H), bet.reshape(1, H))


def _head1_body(hc_ref, w5_ref, e_ref, acc_ref):
    hb = hc_ref[...].astype(jnp.bfloat16)
    wb = w5_ref[...].astype(jnp.bfloat16)
    e = lax.dot_general(hb, wb, (((1,), (1,)), ((), ())),
                        preferred_element_type=jnp.float32)
    e_ref[...] = e

    @pl.when(pl.program_id(0) == 0)
    def _init():
        acc_ref[...] = jnp.zeros_like(acc_ref)

    _df_acc(acc_ref, 0, 1, jnp.sum(e, axis=0, keepdims=True))
    _df_acc(acc_ref, 2, 3, jnp.sum(e * e, axis=0, keepdims=True))


def _head2_body(e_ref, acc_ref, g5_ref, b5_ref, wp1_ref, bp1_ref, wp2_ref,
                bp2_ref, out_ref):
    cnt = float(BN)
    s1 = acc_ref[0:1, :] + acc_ref[1:2, :]
    mean = s1 / cnt
    var = (acc_ref[2:3, :] / cnt - mean * mean) + acc_ref[3:4, :] / cnt
    en = (e_ref[...] - mean) / jnp.sqrt(var + EPS) * g5_ref[...] + b5_ref[...]
    en = jnp.where(en > 0, en, 0.2 * en)
    p = lax.dot_general(en.astype(jnp.bfloat16), wp1_ref[...].astype(jnp.bfloat16),
                        (((1,), (1,)), ((), ())), preferred_element_type=jnp.float32)
    p = jnp.maximum(p + bp1_ref[...], 0.0)
    pb = p.astype(jnp.bfloat16).astype(jnp.float32)
    wb = wp2_ref[...].astype(jnp.bfloat16).astype(jnp.float32)
    o = jnp.sum(pb * wb, axis=1, keepdims=True)
    out_ref[...] = o + bp2_ref[...]


_LAYER_CFG = [
    (7, 16),
    (32, 32),
    (64, 64),
    (64, 64),
]


def kernel(x, fx, W1, g1, b1, W2, g2, b2, W3, g3, b3, W4, g4, b4, W5, g5, b5,
           Wp1, bp1, Wp2, bp2):
    h = jnp.concatenate([x, fx], axis=-1)
    feats = h
    outs = []
    for (CC, CP), W, g, b in zip(_LAYER_CFG, (W1, W2, W3, W4), (g1, g2, g3, g4),
                                 (b1, b2, b3, b4)):
        H = W.shape[0]
        pd, tk, gc = _pd_pallas(feats)
        ftp = jnp.pad(feats.reshape(BN, CC), ((0, 0), (0, GP - CC)))
        G = _make_sc_select_gather(CP)(pd, tk, gc, ftp)
        M, mn, acc = _make_conv(CC, H)(G, ftp, W)
        xo = _finalize(M, mn, acc, g, b, float(BN * K))
        outs.append(xo)
        feats = xo.reshape(B, N, H)

    hc = jnp.concatenate(outs, axis=1)
    e, acc = pl.pallas_call(
        _head1_body,
        grid=(BN // BRC,),
        in_specs=[
            pl.BlockSpec((BRC, 288), lambda r: (r, 0)),
            pl.BlockSpec((256, 288), lambda r: (0, 0)),
        ],
        out_specs=[
            pl.BlockSpec((BRC, 256), lambda r: (r, 0)),
            pl.BlockSpec((4, 256), lambda r: (0, 0)),
        ],
        out_shape=[
            jax.ShapeDtypeStruct((BN, 256), jnp.float32),
            jax.ShapeDtypeStruct((4, 256), jnp.float32),
        ],
    )(hc, W5)

    out = pl.pallas_call(
        _head2_body,
        grid=(BN // BRC,),
        in_specs=[
            pl.BlockSpec((BRC, 256), lambda r: (r, 0)),
            pl.BlockSpec((4, 256), lambda r: (0, 0)),
            pl.BlockSpec((1, 256), lambda r: (0, 0)),
            pl.BlockSpec((1, 256), lambda r: (0, 0)),
            pl.BlockSpec((64, 256), lambda r: (0, 0)),
            pl.BlockSpec((1, 64), lambda r: (0, 0)),
            pl.BlockSpec((1, 64), lambda r: (0, 0)),
            pl.BlockSpec((1, 1), lambda r: (0, 0)),
        ],
        out_specs=pl.BlockSpec((BRC, 1), lambda r: (r, 0)),
        out_shape=jax.ShapeDtypeStruct((BN, 1), jnp.float32),
    )(e, acc, g5.reshape(1, 256), b5.reshape(1, 256), Wp1,
      bp1.reshape(1, 64), Wp2, bp2.reshape(1, 1))

    return out.reshape(B, N, 1)
